# baseline (speedup 1.0000x reference)
from functools import partial

import jax
import jax.numpy as jnp
from jax import lax
from jax.experimental import pallas as pl
from jax.experimental.pallas import tpu as pltpu
from jax.experimental.pallas import tpu_sc as plsc

_N = 100000
_NUM_TYPES = 128

_K = 48000

_WINDOW = 128
_NUM_CORES = 2
_NUM_SUBCORES = 16
_NW = _NUM_CORES * _NUM_SUBCORES
_BPW = 1504
_BPW_LAST = _K - (_NW - 1) * _BPW
_NFULL = _BPW // _WINDOW
_NFULL_LAST = _BPW_LAST // _WINDOW
_TAIL = _BPW - _NFULL * _WINDOW
_TAIL_LAST = _BPW_LAST - _NFULL_LAST * _WINDOW

_NB = 4000
_G = (_N - _K) // _NB
_TC_OFF = _K // _NB


def _sc_gather_fn(embed_dim: int):
    mesh = plsc.VectorSubcoreMesh(core_axis_name="core", subcore_axis_name="subcore")

    @partial(
        pl.kernel,
        out_type=jax.ShapeDtypeStruct((_K, embed_dim), jnp.float32),
        mesh=mesh,
        scratch_types=[
            pltpu.VMEM((_BPW,), jnp.int32),
            pltpu.VMEM((_BPW, embed_dim), jnp.float32),
            pltpu.SemaphoreType.DMA,
            pltpu.SemaphoreType.DMA,
        ],
        compiler_params=pltpu.CompilerParams(use_tc_tiling_on_sc=False),
    )
    def gather(w_hbm, i_hbm, o_hbm, idx_v, rows_v, sem_i, sem_g):
        wid = lax.axis_index("subcore") * _NUM_CORES + lax.axis_index("core")
        base = wid * _BPW
        is_last = wid == _NW - 1

        @pl.when(~is_last)
        def _():
            pltpu.async_copy(i_hbm.at[pl.ds(base, _BPW)], idx_v, sem_i).wait()

        @pl.when(is_last)
        def _():
            pltpu.async_copy(
                i_hbm.at[pl.ds(base, _BPW_LAST)],
                idx_v.at[pl.ds(0, _BPW_LAST)],
                sem_i,
            ).wait()

        nfull = lax.select(is_last, _NFULL_LAST, _NFULL)

        @pl.loop(0, nfull)
        def _(j):
            pltpu.async_copy(
                w_hbm.at[idx_v.at[pl.ds(j * _WINDOW, _WINDOW)]],
                rows_v.at[pl.ds(j * _WINDOW, _WINDOW)],
                sem_g,
            )

        @pl.when(~is_last)
        def _():
            pltpu.async_copy(
                w_hbm.at[idx_v.at[pl.ds(_NFULL * _WINDOW, _TAIL)]],
                rows_v.at[pl.ds(_NFULL * _WINDOW, _TAIL)],
                sem_g,
            )
            pltpu.make_async_copy(o_hbm.at[pl.ds(base, _BPW)], rows_v, sem_g).wait()
            pltpu.sync_copy(rows_v, o_hbm.at[pl.ds(base, _BPW)])

        @pl.when(is_last)
        def _():
            pltpu.async_copy(
                w_hbm.at[idx_v.at[pl.ds(_NFULL_LAST * _WINDOW, _TAIL_LAST)]],
                rows_v.at[pl.ds(_NFULL_LAST * _WINDOW, _TAIL_LAST)],
                sem_g,
            )
            pltpu.make_async_copy(
                o_hbm.at[pl.ds(base, _BPW_LAST)],
                rows_v.at[pl.ds(0, _BPW_LAST)],
                sem_g,
            ).wait()
            pltpu.sync_copy(
                rows_v.at[pl.ds(0, _BPW_LAST)], o_hbm.at[pl.ds(base, _BPW_LAST)]
            )

    return gather


def _tc_body(x_ref, w_ref, o_ref):
    idx = x_ref[...]
    iota = lax.broadcasted_iota(jnp.int32, (_NB, _NUM_TYPES), 1)
    oh = (idx == iota).astype(jnp.bfloat16)
    w = w_ref[...].astype(jnp.bfloat16)
    o_ref[...] = jnp.dot(oh, w, preferred_element_type=jnp.float32)


def kernel(node_type, weight):
    embed_dim = weight.shape[1]
    idx = node_type.reshape(-1)
    sc_out = _sc_gather_fn(embed_dim)(weight, idx)
    tc_out = pl.pallas_call(
        _tc_body,
        grid=(_G,),
        in_specs=[
            pl.BlockSpec((_NB, 1), lambda i: (i + _TC_OFF, 0)),
            pl.BlockSpec((_NUM_TYPES, embed_dim), lambda i: (0, 0)),
        ],
        out_specs=pl.BlockSpec((_NB, embed_dim), lambda i: (i, 0)),
        out_shape=jax.ShapeDtypeStruct((_N - _K, embed_dim), jnp.float32),
    )(node_type, weight)
    return jnp.concatenate([sc_out, tc_out], axis=0)


# speedup vs baseline: 1.0143x; 1.0143x over previous
from functools import partial

import jax
import jax.numpy as jnp
from jax import lax
from jax.experimental import pallas as pl
from jax.experimental.pallas import tpu as pltpu
from jax.experimental.pallas import tpu_sc as plsc

_N = 100000
_NUM_TYPES = 128

_K = 64000

_WINDOW = 128
_NUM_CORES = 2
_NUM_SUBCORES = 16
_NW = _NUM_CORES * _NUM_SUBCORES
_BPW = 2000
_BPW_LAST = _K - (_NW - 1) * _BPW
_NFULL = _BPW // _WINDOW
_NFULL_LAST = _BPW_LAST // _WINDOW
_TAIL = _BPW - _NFULL * _WINDOW
_TAIL_LAST = _BPW_LAST - _NFULL_LAST * _WINDOW

_NB = 4000
_G = (_N - _K) // _NB
_TC_OFF = _K // _NB


def _sc_gather_fn(embed_dim: int):
    mesh = plsc.VectorSubcoreMesh(core_axis_name="core", subcore_axis_name="subcore")

    @partial(
        pl.kernel,
        out_type=jax.ShapeDtypeStruct((_K, embed_dim), jnp.float32),
        mesh=mesh,
        scratch_types=[
            pltpu.VMEM((_BPW,), jnp.int32),
            pltpu.VMEM((_BPW, embed_dim), jnp.float32),
            pltpu.SemaphoreType.DMA,
            pltpu.SemaphoreType.DMA,
        ],
        compiler_params=pltpu.CompilerParams(use_tc_tiling_on_sc=False),
    )
    def gather(w_hbm, i_hbm, o_hbm, idx_v, rows_v, sem_i, sem_g):
        wid = lax.axis_index("subcore") * _NUM_CORES + lax.axis_index("core")
        base = wid * _BPW
        is_last = wid == _NW - 1

        @pl.when(~is_last)
        def _():
            pltpu.async_copy(i_hbm.at[pl.ds(base, _BPW)], idx_v, sem_i).wait()

        @pl.when(is_last)
        def _():
            pltpu.async_copy(
                i_hbm.at[pl.ds(base, _BPW_LAST)],
                idx_v.at[pl.ds(0, _BPW_LAST)],
                sem_i,
            ).wait()

        nfull = lax.select(is_last, _NFULL_LAST, _NFULL)

        @pl.loop(0, nfull)
        def _(j):
            pltpu.async_copy(
                w_hbm.at[idx_v.at[pl.ds(j * _WINDOW, _WINDOW)]],
                rows_v.at[pl.ds(j * _WINDOW, _WINDOW)],
                sem_g,
            )

        @pl.when(~is_last)
        def _():
            pltpu.async_copy(
                w_hbm.at[idx_v.at[pl.ds(_NFULL * _WINDOW, _TAIL)]],
                rows_v.at[pl.ds(_NFULL * _WINDOW, _TAIL)],
                sem_g,
            )
            pltpu.make_async_copy(o_hbm.at[pl.ds(base, _BPW)], rows_v, sem_g).wait()
            pltpu.sync_copy(rows_v, o_hbm.at[pl.ds(base, _BPW)])

        @pl.when(is_last)
        def _():
            pltpu.async_copy(
                w_hbm.at[idx_v.at[pl.ds(_NFULL_LAST * _WINDOW, _TAIL_LAST)]],
                rows_v.at[pl.ds(_NFULL_LAST * _WINDOW, _TAIL_LAST)],
                sem_g,
            )
            pltpu.make_async_copy(
                o_hbm.at[pl.ds(base, _BPW_LAST)],
                rows_v.at[pl.ds(0, _BPW_LAST)],
                sem_g,
            ).wait()
            pltpu.sync_copy(
                rows_v.at[pl.ds(0, _BPW_LAST)], o_hbm.at[pl.ds(base, _BPW_LAST)]
            )

    return gather


def _tc_body(x_ref, w_ref, o_ref):
    idx = x_ref[...]
    iota = lax.broadcasted_iota(jnp.int32, (_NB, _NUM_TYPES), 1)
    oh = (idx == iota).astype(jnp.bfloat16)
    w = w_ref[...].astype(jnp.bfloat16)
    o_ref[...] = jnp.dot(oh, w, preferred_element_type=jnp.float32)


def kernel(node_type, weight):
    embed_dim = weight.shape[1]
    idx = node_type.reshape(-1)
    sc_out = _sc_gather_fn(embed_dim)(weight, idx)
    tc_out = pl.pallas_call(
        _tc_body,
        grid=(_G,),
        in_specs=[
            pl.BlockSpec((_NB, 1), lambda i: (i + _TC_OFF, 0)),
            pl.BlockSpec((_NUM_TYPES, embed_dim), lambda i: (0, 0)),
        ],
        out_specs=pl.BlockSpec((_NB, embed_dim), lambda i: (i, 0)),
        out_shape=jax.ShapeDtypeStruct((_N - _K, embed_dim), jnp.float32),
    )(node_type, weight)
    return jnp.concatenate([sc_out, tc_out], axis=0)


# speedup vs baseline: 1.1795x; 1.1629x over previous
from functools import partial

import jax
import jax.numpy as jnp
from jax import lax
from jax.experimental import pallas as pl
from jax.experimental.pallas import tpu as pltpu
from jax.experimental.pallas import tpu_sc as plsc

_WINDOW = 128
_NUM_CORES = 2
_NUM_SUBCORES = 16
_NW = _NUM_CORES * _NUM_SUBCORES
_N = 100000
_BPW = 3128
_BPW_LAST = _N - (_NW - 1) * _BPW
_TAIL = _BPW - 24 * _WINDOW
_TAIL_LAST = _BPW_LAST - 23 * _WINDOW


def _gather_fn(embed_dim: int):
    mesh = plsc.VectorSubcoreMesh(core_axis_name="core", subcore_axis_name="subcore")

    @partial(
        pl.kernel,
        out_type=jax.ShapeDtypeStruct((_N, embed_dim), jnp.float32),
        mesh=mesh,
        scratch_types=[
            pltpu.VMEM((_BPW,), jnp.int32),
            pltpu.VMEM((_BPW, embed_dim), jnp.float32),
            pltpu.SemaphoreType.DMA,
            pltpu.SemaphoreType.DMA,
        ],
        compiler_params=pltpu.CompilerParams(use_tc_tiling_on_sc=False),
    )
    def gather(w_hbm, i_hbm, o_hbm, idx_v, rows_v, sem_i, sem_g):
        wid = lax.axis_index("subcore") * _NUM_CORES + lax.axis_index("core")
        base = wid * _BPW
        is_last = wid == _NW - 1

        @pl.when(~is_last)
        def _():
            pltpu.async_copy(i_hbm.at[pl.ds(base, _BPW)], idx_v, sem_i).wait()

        @pl.when(is_last)
        def _():
            pltpu.async_copy(
                i_hbm.at[pl.ds(base, _BPW_LAST)],
                idx_v.at[pl.ds(0, _BPW_LAST)],
                sem_i,
            ).wait()

        nfull = lax.select(is_last, 23, 24)

        @pl.loop(0, nfull)
        def _(j):
            pltpu.async_copy(
                w_hbm.at[idx_v.at[pl.ds(j * _WINDOW, _WINDOW)]],
                rows_v.at[pl.ds(j * _WINDOW, _WINDOW)],
                sem_g,
            )

        @pl.when(~is_last)
        def _():
            pltpu.async_copy(
                w_hbm.at[idx_v.at[pl.ds(24 * _WINDOW, _TAIL)]],
                rows_v.at[pl.ds(24 * _WINDOW, _TAIL)],
                sem_g,
            )
            pltpu.make_async_copy(o_hbm.at[pl.ds(base, _BPW)], rows_v, sem_g).wait()
            pltpu.sync_copy(rows_v, o_hbm.at[pl.ds(base, _BPW)])

        @pl.when(is_last)
        def _():
            pltpu.async_copy(
                w_hbm.at[idx_v.at[pl.ds(23 * _WINDOW, _TAIL_LAST)]],
                rows_v.at[pl.ds(23 * _WINDOW, _TAIL_LAST)],
                sem_g,
            )
            pltpu.make_async_copy(
                o_hbm.at[pl.ds(base, _BPW_LAST)],
                rows_v.at[pl.ds(0, _BPW_LAST)],
                sem_g,
            ).wait()
            pltpu.sync_copy(
                rows_v.at[pl.ds(0, _BPW_LAST)], o_hbm.at[pl.ds(base, _BPW_LAST)]
            )

    return gather


def kernel(node_type, weight):
    idx = node_type.reshape(-1)
    return _gather_fn(weight.shape[1])(weight, idx)
